# baseline (speedup 1.0000x reference)
import functools

import jax
import jax.numpy as jnp
from jax import lax
from jax.experimental import pallas as pl
from jax.experimental.pallas import tpu as pltpu
from jax.experimental.pallas import tpu_sc as plsc

N = 10000
D = 128
E = 320000
DA = 144

NC = 2
NS = 16
NW = NC * NS
EPW = E // NW
K = 128
KE = 125
STEPS = EPW // KE
NP = 10240
NPP = NP + 8
RPT = NP // NS

_mesh = plsc.VectorSubcoreMesh(core_axis_name="c", subcore_axis_name="s")


def _make_agg(width):
    @functools.partial(
        pl.kernel,
        out_type=jax.ShapeDtypeStruct((NC * NP, width), jnp.float32),
        mesh=_mesh,
        compiler_params=pltpu.CompilerParams(use_tc_tiling_on_sc=False),
        scratch_types=(
            pltpu.VMEM((K,), jnp.int32),
            pltpu.VMEM((K,), jnp.int32),
            pltpu.VMEM((K,), jnp.int32),
            pltpu.VMEM((K,), jnp.int32),
            pltpu.VMEM((K, width), jnp.float32),
            pltpu.VMEM((K, width), jnp.float32),
            pltpu.VMEM_SHARED((NPP, width), jnp.float32),
            pltpu.SemaphoreType.DMA,
            pltpu.SemaphoreType.DMA,
            pltpu.SemaphoreType.DMA,
            pltpu.SemaphoreType.DMA,
        ),
    )
    def _agg(x_hbm, src_hbm, dst_hbm, zrow_hbm, agg_out,
             srcv0, dstv0, srcv1, dstv1, rows0, rows1, agg_sp,
             gsem0, gsem1, ssem0, ssem1):
        c = lax.axis_index("c")
        s = lax.axis_index("s")
        wid = c * NS + s
        row0 = wid * STEPS
        r0 = s * RPT
        pltpu.sync_copy(zrow_hbm, agg_sp.at[pl.ds(r0, RPT)])
        plsc.subcore_barrier()

        def load_idx(i, srcv, dstv):
            pltpu.sync_copy(src_hbm.at[row0 + i], srcv)
            pltpu.sync_copy(dst_hbm.at[row0 + i], dstv)

        def gather(srcv, rows, gsem):
            return pltpu.async_copy(x_hbm.at[srcv], rows, gsem)

        def gwait(srcv, rows, gsem):
            pltpu.make_async_copy(x_hbm.at[srcv], rows, gsem).wait()

        def scatter(dstv, rows, ssem):
            return pltpu.async_copy(rows, agg_sp.at[dstv], ssem, add=True)

        load_idx(0, srcv0, dstv0)
        gather(srcv0, rows0, gsem0)
        load_idx(1, srcv1, dstv1)
        gather(srcv1, rows1, gsem1)

        @pl.loop(0, STEPS - 2, step=2)
        def _steady(i0):
            gwait(srcv0, rows0, gsem0)
            s0 = scatter(dstv0, rows0, ssem0)
            gwait(srcv1, rows1, gsem1)
            s1 = scatter(dstv1, rows1, ssem1)
            s0.wait()
            load_idx(i0 + 2, srcv0, dstv0)
            gather(srcv0, rows0, gsem0)
            s1.wait()
            load_idx(i0 + 3, srcv1, dstv1)
            gather(srcv1, rows1, gsem1)

        gwait(srcv0, rows0, gsem0)
        s0 = scatter(dstv0, rows0, ssem0)
        gwait(srcv1, rows1, gsem1)
        s1 = scatter(dstv1, rows1, ssem1)
        s0.wait()
        s1.wait()

        plsc.subcore_barrier()
        pltpu.sync_copy(agg_sp.at[pl.ds(r0, RPT)],
                        agg_out.at[pl.ds(c * NP + r0, RPT)])

    return _agg


_agg_aug = _make_agg(DA)
_agg_plain = _make_agg(D)


BR = 1000
GRID = N // BR

_row_spec = pl.BlockSpec((BR, D), lambda i: (i, 0))
_aug_spec = pl.BlockSpec((BR, DA), lambda i: (i, 0))
_w_spec = pl.BlockSpec((D, D), lambda i: (0, 0))
_b_spec = pl.BlockSpec((1, D), lambda i: (0, 0))


def _tc1_body(x_ref, a0_ref, a1_ref, w_ref, b_ref, h_ref, recip_ref):
    deg = a0_ref[:, D:D + 1] + a1_ref[:, D:D + 1]
    recip = 1.0 / jnp.maximum(deg, 1.0)
    agg = a0_ref[:, :D] + a1_ref[:, :D]
    rst = x_ref[...] + agg * recip
    h_ref[...] = jnp.maximum(
        jnp.dot(rst, w_ref[...], preferred_element_type=jnp.float32)
        + b_ref[...], 0.0)
    recip_ref[...] = recip + jnp.zeros((BR, D), jnp.float32)


_tc1 = pl.pallas_call(
    _tc1_body,
    grid=(GRID,),
    in_specs=[_row_spec, _aug_spec, _aug_spec, _w_spec, _b_spec],
    out_specs=(_row_spec, _row_spec),
    out_shape=(jax.ShapeDtypeStruct((N, D), jnp.float32),
               jax.ShapeDtypeStruct((N, D), jnp.float32)),
)


def _tc2_body(x_ref, a0_ref, a1_ref, recip_ref, w_ref, b_ref,
              wd1_ref, bd1_ref, wd2_ref, bd2_ref,
              hg_ref, rec_ref, acc_ref):
    i = pl.program_id(0)
    rst = x_ref[...] + (a0_ref[...] + a1_ref[...]) * recip_ref[...]
    h2 = jnp.maximum(
        jnp.dot(rst, w_ref[...], preferred_element_type=jnp.float32)
        + b_ref[...], 0.0)
    part = jnp.sum(h2, axis=0, keepdims=True)

    @pl.when(i == 0)
    def _():
        acc_ref[...] = part

    @pl.when(i > 0)
    def _():
        acc_ref[...] = acc_ref[...] + part

    @pl.when(i == GRID - 1)
    def _():
        hg = acc_ref[...] * (1.0 / N)
        hg_ref[...] = hg
        r1 = jnp.maximum(
            jnp.dot(hg, wd1_ref[...], preferred_element_type=jnp.float32)
            + bd1_ref[...], 0.0)
        rec_ref[...] = (
            jnp.dot(r1, wd2_ref[...], preferred_element_type=jnp.float32)
            + bd2_ref[...])


_tc2 = pl.pallas_call(
    _tc2_body,
    grid=(GRID,),
    in_specs=[_row_spec, _row_spec, _row_spec, _row_spec,
              _w_spec, _b_spec, _w_spec, _b_spec, _w_spec, _b_spec],
    out_specs=(_b_spec, _b_spec),
    out_shape=(jax.ShapeDtypeStruct((1, D), jnp.float32),
               jax.ShapeDtypeStruct((1, D), jnp.float32)),
    scratch_shapes=[pltpu.VMEM((1, D), jnp.float32)],
)


def kernel(features, edge_index, W1, b1, W2, b2, Wd1, bd1, Wd2, bd2):
    ei = edge_index.reshape(2, NW, STEPS, KE)
    pad = ((0, 0), (0, 0), (0, K - KE))
    srcm = jnp.pad(ei[0], pad).reshape(NW * STEPS, K)
    dstm = jnp.pad(ei[1], pad, constant_values=NP).reshape(NW * STEPS, K)
    x_aug = jnp.concatenate(
        [features, jnp.ones((N, 1), jnp.float32),
         jnp.zeros((N, DA - D - 1), jnp.float32)], axis=1)
    zaug = jnp.zeros((RPT, DA), jnp.float32)
    zrow = jnp.zeros((RPT, D), jnp.float32)

    aggp = _agg_aug(x_aug, srcm, dstm, zaug)
    h1, recip = _tc1(features, aggp[:N], aggp[NP:NP + N], W1,
                     b1.reshape(1, D))
    aggp2 = _agg_plain(h1, srcm, dstm, zrow)
    hg, rec = _tc2(h1, aggp2[:N], aggp2[NP:NP + N], recip, W2,
                   b2.reshape(1, D), Wd1, bd1.reshape(1, D),
                   Wd2, bd2.reshape(1, D))
    return (hg, rec)


# speedup vs baseline: 1.0196x; 1.0196x over previous
import functools

import jax
import jax.numpy as jnp
from jax import lax
from jax.experimental import pallas as pl
from jax.experimental.pallas import tpu as pltpu
from jax.experimental.pallas import tpu_sc as plsc

N = 10000
D = 128
E = 320000
DA = 144

NC = 2
NS = 16
NW = NC * NS
EPW = E // NW
K = 128
KE = 125
STEPS = EPW // KE
NP = 10240
NPP = NP + NW
RPT = NP // NS

_mesh = plsc.VectorSubcoreMesh(core_axis_name="c", subcore_axis_name="s")


def _make_agg(width):
    @functools.partial(
        pl.kernel,
        out_type=jax.ShapeDtypeStruct((NC * NP, width), jnp.float32),
        mesh=_mesh,
        compiler_params=pltpu.CompilerParams(use_tc_tiling_on_sc=False),
        scratch_types=(
            pltpu.VMEM((2, K), jnp.int32),
            pltpu.VMEM((2, K), jnp.int32),
            pltpu.VMEM((K, width), jnp.float32),
            pltpu.VMEM((K, width), jnp.float32),
            pltpu.VMEM_SHARED((NPP, width), jnp.float32),
            pltpu.SemaphoreType.DMA,
            pltpu.SemaphoreType.DMA,
            pltpu.SemaphoreType.DMA,
            pltpu.SemaphoreType.DMA,
        ),
    )
    def _agg(x_hbm, idx_hbm, zrow_hbm, agg_out,
             idx0, idx1, rows0, rows1, agg_sp,
             gsem0, gsem1, ssem0, ssem1):
        c = lax.axis_index("c")
        s = lax.axis_index("s")
        wid = c * NS + s
        row0 = wid * STEPS
        r0 = s * RPT
        pltpu.sync_copy(zrow_hbm, agg_sp.at[pl.ds(r0, RPT)])
        plsc.subcore_barrier()

        def load_idx(i, idxv):
            pltpu.sync_copy(idx_hbm.at[row0 + i], idxv)

        def gather(idxv, rows, gsem):
            return pltpu.async_copy(x_hbm.at[idxv.at[0]], rows, gsem)

        def gwait(idxv, rows, gsem):
            pltpu.make_async_copy(x_hbm.at[idxv.at[0]], rows, gsem).wait()

        def scatter(idxv, rows, ssem):
            return pltpu.async_copy(rows, agg_sp.at[idxv.at[1]], ssem,
                                    add=True)

        load_idx(0, idx0)
        gather(idx0, rows0, gsem0)
        load_idx(1, idx1)
        gather(idx1, rows1, gsem1)

        @pl.loop(0, STEPS - 2, step=2)
        def _steady(i0):
            gwait(idx0, rows0, gsem0)
            s0 = scatter(idx0, rows0, ssem0)
            gwait(idx1, rows1, gsem1)
            s1 = scatter(idx1, rows1, ssem1)
            s0.wait()
            load_idx(i0 + 2, idx0)
            gather(idx0, rows0, gsem0)
            s1.wait()
            load_idx(i0 + 3, idx1)
            gather(idx1, rows1, gsem1)

        gwait(idx0, rows0, gsem0)
        s0 = scatter(idx0, rows0, ssem0)
        gwait(idx1, rows1, gsem1)
        s1 = scatter(idx1, rows1, ssem1)
        s0.wait()
        s1.wait()

        plsc.subcore_barrier()
        pltpu.sync_copy(agg_sp.at[pl.ds(r0, RPT)],
                        agg_out.at[pl.ds(c * NP + r0, RPT)])

    return _agg


_agg_aug = _make_agg(DA)
_agg_plain = _make_agg(D)


BR = 1000
GRID = N // BR

_row_spec = pl.BlockSpec((BR, D), lambda i: (i, 0))
_aug_spec = pl.BlockSpec((BR, DA), lambda i: (i, 0))
_w_spec = pl.BlockSpec((D, D), lambda i: (0, 0))
_b_spec = pl.BlockSpec((1, D), lambda i: (0, 0))


def _tc1_body(x_ref, a0_ref, a1_ref, w_ref, b_ref, h_ref, recip_ref):
    deg = a0_ref[:, D:D + 1] + a1_ref[:, D:D + 1]
    recip = 1.0 / jnp.maximum(deg, 1.0)
    agg = a0_ref[:, :D] + a1_ref[:, :D]
    rst = x_ref[...] + agg * recip
    h_ref[...] = jnp.maximum(
        jnp.dot(rst, w_ref[...], preferred_element_type=jnp.float32)
        + b_ref[...], 0.0)
    recip_ref[...] = recip + jnp.zeros((BR, D), jnp.float32)


_tc1 = pl.pallas_call(
    _tc1_body,
    grid=(GRID,),
    in_specs=[_row_spec, _aug_spec, _aug_spec, _w_spec, _b_spec],
    out_specs=(_row_spec, _row_spec),
    out_shape=(jax.ShapeDtypeStruct((N, D), jnp.float32),
               jax.ShapeDtypeStruct((N, D), jnp.float32)),
)


def _tc2_body(x_ref, a0_ref, a1_ref, recip_ref, w_ref, b_ref,
              wd1_ref, bd1_ref, wd2_ref, bd2_ref,
              hg_ref, rec_ref, acc_ref):
    i = pl.program_id(0)
    rst = x_ref[...] + (a0_ref[...] + a1_ref[...]) * recip_ref[...]
    h2 = jnp.maximum(
        jnp.dot(rst, w_ref[...], preferred_element_type=jnp.float32)
        + b_ref[...], 0.0)
    part = jnp.sum(h2, axis=0, keepdims=True)

    @pl.when(i == 0)
    def _():
        acc_ref[...] = part

    @pl.when(i > 0)
    def _():
        acc_ref[...] = acc_ref[...] + part

    @pl.when(i == GRID - 1)
    def _():
        hg = acc_ref[...] * (1.0 / N)
        hg_ref[...] = hg
        r1 = jnp.maximum(
            jnp.dot(hg, wd1_ref[...], preferred_element_type=jnp.float32)
            + bd1_ref[...], 0.0)
        rec_ref[...] = (
            jnp.dot(r1, wd2_ref[...], preferred_element_type=jnp.float32)
            + bd2_ref[...])


_tc2 = pl.pallas_call(
    _tc2_body,
    grid=(GRID,),
    in_specs=[_row_spec, _row_spec, _row_spec, _row_spec,
              _w_spec, _b_spec, _w_spec, _b_spec, _w_spec, _b_spec],
    out_specs=(_b_spec, _b_spec),
    out_shape=(jax.ShapeDtypeStruct((1, D), jnp.float32),
               jax.ShapeDtypeStruct((1, D), jnp.float32)),
    scratch_shapes=[pltpu.VMEM((1, D), jnp.float32)],
)


def kernel(features, edge_index, W1, b1, W2, b2, Wd1, bd1, Wd2, bd2):
    ei = edge_index.reshape(2, NW, STEPS, KE)
    srcp = jnp.pad(ei[0], ((0, 0), (0, 0), (0, K - KE)))
    padrow = jnp.broadcast_to(
        (NP + jnp.arange(NW, dtype=jnp.int32))[:, None, None],
        (NW, STEPS, K - KE))
    dstp = jnp.concatenate([ei[1], padrow], axis=2)
    idxm = jnp.stack([srcp, dstp], axis=2).reshape(NW * STEPS, 2, K)
    x_aug = jnp.concatenate(
        [features, jnp.ones((N, 1), jnp.float32),
         jnp.zeros((N, DA - D - 1), jnp.float32)], axis=1)
    zaug = jnp.zeros((RPT, DA), jnp.float32)
    zrow = jnp.zeros((RPT, D), jnp.float32)

    aggp = _agg_aug(x_aug, idxm, zaug)
    h1, recip = _tc1(features, aggp[:N], aggp[NP:NP + N], W1,
                     b1.reshape(1, D))
    aggp2 = _agg_plain(h1, idxm, zrow)
    hg, rec = _tc2(h1, aggp2[:N], aggp2[NP:NP + N], recip, W2,
                   b2.reshape(1, D), Wd1, bd1.reshape(1, D),
                   Wd2, bd2.reshape(1, D))
    return (hg, rec)


# speedup vs baseline: 2.1364x; 2.0953x over previous
import functools

import jax
import jax.numpy as jnp
from jax import lax
from jax.experimental import pallas as pl
from jax.experimental.pallas import tpu as pltpu
from jax.experimental.pallas import tpu_sc as plsc

N = 10000
D = 128
E = 320000
DA = 144

NC = 2
NS = 16
NW = NC * NS
EPW = E // NW
K = 100
KE = 100
STEPS = EPW // KE
NP = 10240
NPP = NP + NW
RPT = NP // NS

_mesh = plsc.VectorSubcoreMesh(core_axis_name="c", subcore_axis_name="s")


def _make_agg(width):
    @functools.partial(
        pl.kernel,
        out_type=jax.ShapeDtypeStruct((NC * NP, width), jnp.float32),
        mesh=_mesh,
        compiler_params=pltpu.CompilerParams(use_tc_tiling_on_sc=False),
        scratch_types=(
            pltpu.VMEM((2, K), jnp.int32),
            pltpu.VMEM((2, K), jnp.int32),
            pltpu.VMEM((K, width), jnp.float32),
            pltpu.VMEM((K, width), jnp.float32),
            pltpu.VMEM_SHARED((NPP, width), jnp.float32),
            pltpu.SemaphoreType.DMA,
            pltpu.SemaphoreType.DMA,
            pltpu.SemaphoreType.DMA,
            pltpu.SemaphoreType.DMA,
        ),
    )
    def _agg(x_hbm, idx_hbm, zrow_hbm, agg_out,
             idx0, idx1, rows0, rows1, agg_sp,
             gsem0, gsem1, ssem0, ssem1):
        c = lax.axis_index("c")
        s = lax.axis_index("s")
        wid = c * NS + s
        row0 = wid * STEPS
        r0 = s * RPT
        pltpu.sync_copy(zrow_hbm, agg_sp.at[pl.ds(r0, RPT)])
        plsc.subcore_barrier()

        def load_idx(i, idxv):
            pltpu.sync_copy(idx_hbm.at[row0 + i], idxv)

        def gather(idxv, rows, gsem):
            return pltpu.async_copy(x_hbm.at[idxv.at[0]], rows, gsem)

        def gwait(idxv, rows, gsem):
            pltpu.make_async_copy(x_hbm.at[idxv.at[0]], rows, gsem).wait()

        def scatter(idxv, rows, ssem):
            return pltpu.async_copy(rows, agg_sp.at[idxv.at[1]], ssem,
                                    add=True)

        load_idx(0, idx0)
        gather(idx0, rows0, gsem0)
        load_idx(1, idx1)
        gather(idx1, rows1, gsem1)

        @pl.loop(0, STEPS - 2, step=2)
        def _steady(i0):
            gwait(idx0, rows0, gsem0)
            s0 = scatter(idx0, rows0, ssem0)
            gwait(idx1, rows1, gsem1)
            s1 = scatter(idx1, rows1, ssem1)
            s0.wait()
            load_idx(i0 + 2, idx0)
            gather(idx0, rows0, gsem0)
            s1.wait()
            load_idx(i0 + 3, idx1)
            gather(idx1, rows1, gsem1)

        gwait(idx0, rows0, gsem0)
        s0 = scatter(idx0, rows0, ssem0)
        gwait(idx1, rows1, gsem1)
        s1 = scatter(idx1, rows1, ssem1)
        s0.wait()
        s1.wait()

        plsc.subcore_barrier()
        pltpu.sync_copy(agg_sp.at[pl.ds(r0, RPT)],
                        agg_out.at[pl.ds(c * NP + r0, RPT)])

    return _agg


_agg_aug = _make_agg(DA)
_agg_plain = _make_agg(D)


BR = 1000
GRID = N // BR

_row_spec = pl.BlockSpec((BR, D), lambda i: (i, 0))
_aug_spec = pl.BlockSpec((BR, DA), lambda i: (i, 0))
_w_spec = pl.BlockSpec((D, D), lambda i: (0, 0))
_b_spec = pl.BlockSpec((1, D), lambda i: (0, 0))


def _tc1_body(x_ref, a0_ref, a1_ref, w_ref, b_ref, h_ref, recip_ref):
    deg = a0_ref[:, D:D + 1] + a1_ref[:, D:D + 1]
    recip = 1.0 / jnp.maximum(deg, 1.0)
    agg = a0_ref[:, :D] + a1_ref[:, :D]
    rst = x_ref[...] + agg * recip
    h_ref[...] = jnp.maximum(
        jnp.dot(rst, w_ref[...], preferred_element_type=jnp.float32)
        + b_ref[...], 0.0)
    recip_ref[...] = recip + jnp.zeros((BR, D), jnp.float32)


_tc1 = pl.pallas_call(
    _tc1_body,
    grid=(GRID,),
    in_specs=[_row_spec, _aug_spec, _aug_spec, _w_spec, _b_spec],
    out_specs=(_row_spec, _row_spec),
    out_shape=(jax.ShapeDtypeStruct((N, D), jnp.float32),
               jax.ShapeDtypeStruct((N, D), jnp.float32)),
)


def _tc2_body(x_ref, a0_ref, a1_ref, recip_ref, w_ref, b_ref,
              wd1_ref, bd1_ref, wd2_ref, bd2_ref,
              hg_ref, rec_ref, acc_ref):
    i = pl.program_id(0)
    rst = x_ref[...] + (a0_ref[...] + a1_ref[...]) * recip_ref[...]
    h2 = jnp.maximum(
        jnp.dot(rst, w_ref[...], preferred_element_type=jnp.float32)
        + b_ref[...], 0.0)
    part = jnp.sum(h2, axis=0, keepdims=True)

    @pl.when(i == 0)
    def _():
        acc_ref[...] = part

    @pl.when(i > 0)
    def _():
        acc_ref[...] = acc_ref[...] + part

    @pl.when(i == GRID - 1)
    def _():
        hg = acc_ref[...] * (1.0 / N)
        hg_ref[...] = hg
        r1 = jnp.maximum(
            jnp.dot(hg, wd1_ref[...], preferred_element_type=jnp.float32)
            + bd1_ref[...], 0.0)
        rec_ref[...] = (
            jnp.dot(r1, wd2_ref[...], preferred_element_type=jnp.float32)
            + bd2_ref[...])


_tc2 = pl.pallas_call(
    _tc2_body,
    grid=(GRID,),
    in_specs=[_row_spec, _row_spec, _row_spec, _row_spec,
              _w_spec, _b_spec, _w_spec, _b_spec, _w_spec, _b_spec],
    out_specs=(_b_spec, _b_spec),
    out_shape=(jax.ShapeDtypeStruct((1, D), jnp.float32),
               jax.ShapeDtypeStruct((1, D), jnp.float32)),
    scratch_shapes=[pltpu.VMEM((1, D), jnp.float32)],
)


def kernel(features, edge_index, W1, b1, W2, b2, Wd1, bd1, Wd2, bd2):
    ei = edge_index.reshape(2, NW, STEPS, KE)
    idxm = jnp.stack([ei[0], ei[1]], axis=2).reshape(NW * STEPS, 2, K)
    x_aug = jnp.concatenate(
        [features, jnp.ones((N, 1), jnp.float32),
         jnp.zeros((N, DA - D - 1), jnp.float32)], axis=1)
    zaug = jnp.zeros((RPT, DA), jnp.float32)
    zrow = jnp.zeros((RPT, D), jnp.float32)

    aggp = _agg_aug(x_aug, idxm, zaug)
    h1, recip = _tc1(features, aggp[:N], aggp[NP:NP + N], W1,
                     b1.reshape(1, D))
    aggp2 = _agg_plain(h1, idxm, zrow)
    hg, rec = _tc2(h1, aggp2[:N], aggp2[NP:NP + N], recip, W2,
                   b2.reshape(1, D), Wd1, bd1.reshape(1, D),
                   Wd2, bd2.reshape(1, D))
    return (hg, rec)
